# Initial kernel scaffold; baseline (speedup 1.0000x reference)
import jax
import jax.numpy as jnp
from jax.experimental import pallas as pl


def kernel(x, edge_index, rnbrw_weights, W, a_src, a_dst):
    raise NotImplementedError("write your pallas kernel here")


# speedup vs baseline: 6.4341x; 6.4341x over previous
import functools

import jax
import jax.numpy as jnp
from jax import lax
from jax.experimental import pallas as pl
from jax.experimental.pallas import tpu as pltpu
from jax.experimental.pallas import tpu_sc as plsc

N = 10000
E = 320000
D = 128
EPS = 1e-08
NEG_INIT = -3.0e38

NC = 2
NS = 16
NW = NC * NS
EPW = E // NW
L = 16
CHUNK = 400

_MESH = plsc.VectorSubcoreMesh(core_axis_name="c", subcore_axis_name="s")


def _wid():
    return lax.axis_index("s") * NC + lax.axis_index("c")


def _k1_body(x_ref, w_ref, asrc_ref, adst_ref, rn_ref, h_ref, s_ref, b_ref):
    h = lax.dot_general(x_ref[...], w_ref[...],
                        dimension_numbers=(((1,), (1,)), ((), ())),
                        preferred_element_type=jnp.float32)
    h_ref[...] = h
    s_ref[:, 0:1] = jnp.sum(h * asrc_ref[...], axis=1, keepdims=True)
    s_ref[:, 1:2] = jnp.sum(h * adst_ref[...], axis=1, keepdims=True)
    b_ref[...] = jnp.log(rn_ref[...] + EPS)


RN_COLS = 2000


def _k1(x, w, a_src, a_dst, rn2d):
    grid = 10
    bn = N // grid
    br = (E // RN_COLS) // grid
    return pl.pallas_call(
        _k1_body,
        grid=(grid,),
        in_specs=[
            pl.BlockSpec((bn, D), lambda i: (i, 0)),
            pl.BlockSpec((D, D), lambda i: (0, 0)),
            pl.BlockSpec((1, D), lambda i: (0, 0)),
            pl.BlockSpec((1, D), lambda i: (0, 0)),
            pl.BlockSpec((br, RN_COLS), lambda i: (i, 0)),
        ],
        out_specs=[
            pl.BlockSpec((bn, D), lambda i: (i, 0)),
            pl.BlockSpec((bn, 2), lambda i: (i, 0)),
            pl.BlockSpec((br, RN_COLS), lambda i: (i, 0)),
        ],
        out_shape=[
            jax.ShapeDtypeStruct((N, D), jnp.float32),
            jax.ShapeDtypeStruct((N, 2), jnp.float32),
            jax.ShapeDtypeStruct((E // RN_COLS, RN_COLS), jnp.float32),
        ],
    )(x, w, a_src, a_dst, rn2d)


def _k2_body(s_hbm, src_hbm, dst_hbm, bias_hbm, score_hbm, mxp_hbm,
             s_v, src_v, dst_v, bias_v, score_v, mx_v):
    wid = _wid()
    base = wid * EPW
    pltpu.sync_copy(s_hbm, s_v)
    pltpu.sync_copy(src_hbm.at[pl.ds(base, EPW)], src_v)
    pltpu.sync_copy(dst_hbm.at[pl.ds(base, EPW)], dst_v)
    pltpu.sync_copy(bias_hbm.at[pl.ds(base, EPW)], bias_v)

    neg = jnp.full((L,), NEG_INIT, jnp.float32)

    def init_body(i, c):
        mx_v[pl.ds(i * L, L)] = neg
        return c

    lax.fori_loop(0, N // L, init_body, 0)

    def edge_body(i, c):
        sl = pl.ds(i * L, L)
        sv = src_v[sl]
        dv = dst_v[sl]
        se = plsc.load_gather(s_v, [sv * 2])
        de = plsc.load_gather(s_v, [dv * 2 + 1])
        sc = se + de + bias_v[sl]
        sc = jnp.where(sc >= 0.0, sc, 0.2 * sc)
        score_v[sl] = sc

        def cond(p):
            return jnp.any(p)

        def body(p):
            cur = plsc.load_gather(mx_v, [dv])
            plsc.store_scatter(mx_v, [dv], jnp.maximum(cur, sc), mask=p)
            chk = plsc.load_gather(mx_v, [dv])
            return jnp.logical_and(p, chk < sc)

        lax.while_loop(cond, body, jnp.ones((L,), jnp.bool_))
        return c

    lax.fori_loop(0, EPW // L, edge_body, 0)
    pltpu.sync_copy(score_v, score_hbm.at[pl.ds(base, EPW)])
    pltpu.sync_copy(mx_v, mxp_hbm.at[wid])


def _k2(s_flat, src, dst, bias):
    f = pl.kernel(
        _k2_body,
        out_type=[
            jax.ShapeDtypeStruct((E,), jnp.float32),
            jax.ShapeDtypeStruct((NW, N), jnp.float32),
        ],
        mesh=_MESH,
        compiler_params=pltpu.CompilerParams(needs_layout_passes=False),
        scratch_types=[
            pltpu.VMEM((2 * N,), jnp.float32),
            pltpu.VMEM((EPW,), jnp.int32),
            pltpu.VMEM((EPW,), jnp.int32),
            pltpu.VMEM((EPW,), jnp.float32),
            pltpu.VMEM((EPW,), jnp.float32),
            pltpu.VMEM((N,), jnp.float32),
        ],
    )
    return f(s_flat, src, dst, bias)


def _combine(part_hbm, acc_v, row_v, op):
    pltpu.sync_copy(part_hbm.at[0], acc_v)

    def row_body(r, c):
        pltpu.sync_copy(part_hbm.at[r], row_v)

        def vec_body(i, c2):
            sl = pl.ds(i * L, L)
            acc_v[sl] = op(acc_v[sl], row_v[sl])
            return c2

        lax.fori_loop(0, N // L, vec_body, 0)
        return c

    lax.fori_loop(1, NW, row_body, 0)


def _k3_body(mxp_hbm, dst_hbm, score_hbm, ex_hbm, denp_hbm,
             mx_v, row_v, dst_v, score_v, ex_v, den_v):
    wid = _wid()
    base = wid * EPW
    _combine(mxp_hbm, mx_v, row_v, jnp.maximum)
    pltpu.sync_copy(dst_hbm.at[pl.ds(base, EPW)], dst_v)
    pltpu.sync_copy(score_hbm.at[pl.ds(base, EPW)], score_v)

    zero = jnp.zeros((L,), jnp.float32)

    def init_body(i, c):
        den_v[pl.ds(i * L, L)] = zero
        return c

    lax.fori_loop(0, N // L, init_body, 0)

    def edge_body(i, c):
        sl = pl.ds(i * L, L)
        dv = dst_v[sl]
        ex = jnp.exp(score_v[sl] - plsc.load_gather(mx_v, [dv]))
        ex_v[sl] = ex
        plsc.addupdate_scatter(den_v, [dv], ex)
        return c

    lax.fori_loop(0, EPW // L, edge_body, 0)
    pltpu.sync_copy(ex_v, ex_hbm.at[pl.ds(base, EPW)])
    pltpu.sync_copy(den_v, denp_hbm.at[wid])


def _k3(mxp, dst, score):
    f = pl.kernel(
        _k3_body,
        out_type=[
            jax.ShapeDtypeStruct((E,), jnp.float32),
            jax.ShapeDtypeStruct((NW, N), jnp.float32),
        ],
        mesh=_MESH,
        compiler_params=pltpu.CompilerParams(needs_layout_passes=False),
        scratch_types=[
            pltpu.VMEM((N,), jnp.float32),
            pltpu.VMEM((N,), jnp.float32),
            pltpu.VMEM((EPW,), jnp.int32),
            pltpu.VMEM((EPW,), jnp.float32),
            pltpu.VMEM((EPW,), jnp.float32),
            pltpu.VMEM((N,), jnp.float32),
        ],
    )
    return f(mxp, dst, score)


T_BASES = (0, 3328, 6656)
T_SIZES = (3328, 3328, 3344)
ACC_ROWS = max(T_SIZES) + 8
ROWS_PER_TILE = 208


def _k4_body(denp_hbm, dst_hbm, src_hbm, ex_hbm, h_hbm,
             alpha_hbm, outp_hbm,
             den_v, row_v, dst_v, ex_v, rows_v, sidx_v, didx_v,
             sem, acc):
    cid = lax.axis_index("c")
    sid = lax.axis_index("s")
    wid = sid * NC + cid
    base = wid * EPW
    _combine(denp_hbm, den_v, row_v, jnp.add)
    pltpu.sync_copy(dst_hbm.at[pl.ds(base, EPW)], dst_v)
    pltpu.sync_copy(ex_hbm.at[pl.ds(base, EPW)], ex_v)

    def alpha_body(i, c):
        sl = pl.ds(i * L, L)
        dv = dst_v[sl]
        ex_v[sl] = ex_v[sl] / (plsc.load_gather(den_v, [dv]) + EPS)
        return c

    lax.fori_loop(0, EPW // L, alpha_body, 0)
    pltpu.sync_copy(ex_v, alpha_hbm.at[pl.ds(base, EPW)])

    zero = jnp.zeros((L,), jnp.float32)

    def zrow_body(j, c):
        for r in range(D // L):
            rows_v[j, pl.ds(r * L, L)] = zero
        return c

    lax.fori_loop(0, CHUNK, zrow_body, 0)
    lane = lax.iota(jnp.int32, L)

    for tb, ts in zip(T_BASES, T_SIZES):
        pltpu.sync_copy(rows_v.at[pl.ds(0, ROWS_PER_TILE)],
                        acc.at[pl.ds(sid * ROWS_PER_TILE, ROWS_PER_TILE)])

        @pl.when(sid == 0)
        def _zero_tail():
            extra = ts + 8 - NS * ROWS_PER_TILE
            pltpu.sync_copy(rows_v.at[pl.ds(0, extra)],
                            acc.at[pl.ds(NS * ROWS_PER_TILE, extra)])

        plsc.subcore_barrier()

        def chunk_body(c, carry):
            cb = c * CHUNK
            pltpu.sync_copy(src_hbm.at[pl.ds(base + cb, CHUNK)], sidx_v)
            pltpu.sync_copy(dst_hbm.at[pl.ds(base + cb, CHUNK)], didx_v)
            gather = pltpu.async_copy(h_hbm.at[sidx_v], rows_v, sem)

            def remap_body(i, c2):
                sl = pl.ds(i * L, L)
                dv = didx_v[sl]
                loc = dv - tb
                ok = jnp.logical_and(dv >= tb, dv < tb + ts)
                didx_v[sl] = jnp.where(ok, loc, ts + (lane & 7))
                return c2

            lax.fori_loop(0, CHUNK // L, remap_body, 0)
            gather.wait()

            def scale_body(j, c2):
                a_b = plsc.load_gather(
                    ex_v, [jnp.zeros((L,), jnp.int32) + (cb + j)])
                for r in range(D // L):
                    sl = pl.ds(r * L, L)
                    rows_v[j, sl] = rows_v[j, sl] * a_b
                return c2

            lax.fori_loop(0, CHUNK, scale_body, 0)
            pltpu.sync_copy(rows_v, acc.at[didx_v], add=True)
            return carry

        lax.fori_loop(0, EPW // CHUNK, chunk_body, 0)
        plsc.subcore_barrier()

        pltpu.sync_copy(
            acc.at[pl.ds(sid * ROWS_PER_TILE, ROWS_PER_TILE)],
            outp_hbm.at[cid].at[pl.ds(tb + sid * ROWS_PER_TILE,
                                      ROWS_PER_TILE)])

        @pl.when(jnp.logical_and(sid == 0, ts > NS * ROWS_PER_TILE))
        def _dump_tail():
            extra = max(ts - NS * ROWS_PER_TILE, 8)
            pltpu.sync_copy(
                acc.at[pl.ds(NS * ROWS_PER_TILE, extra)],
                outp_hbm.at[cid].at[pl.ds(tb + NS * ROWS_PER_TILE, extra)])

        plsc.subcore_barrier()

        lax.fori_loop(0, CHUNK, zrow_body, 0)


def _k4(denp, dst, src, ex, h):
    f = pl.kernel(
        _k4_body,
        out_type=[
            jax.ShapeDtypeStruct((E,), jnp.float32),
            jax.ShapeDtypeStruct((NC, N, D), jnp.float32),
        ],
        mesh=_MESH,
        compiler_params=pltpu.CompilerParams(needs_layout_passes=False),
        scratch_types=[
            pltpu.VMEM((N,), jnp.float32),
            pltpu.VMEM((N,), jnp.float32),
            pltpu.VMEM((EPW,), jnp.int32),
            pltpu.VMEM((EPW,), jnp.float32),
            pltpu.VMEM((CHUNK, D), jnp.float32),
            pltpu.VMEM((CHUNK,), jnp.int32),
            pltpu.VMEM((CHUNK,), jnp.int32),
            pltpu.SemaphoreType.DMA,
            pltpu.VMEM_SHARED((ACC_ROWS, D), jnp.float32),
        ],
    )
    return f(denp, dst, src, ex, h)


def _k5_body(a_ref, b_ref, o_ref):
    o_ref[...] = a_ref[...] + b_ref[...]


def _k5(p0, p1):
    grid = 10
    bn = N // grid
    return pl.pallas_call(
        _k5_body,
        grid=(grid,),
        in_specs=[
            pl.BlockSpec((bn, D), lambda i: (i, 0)),
            pl.BlockSpec((bn, D), lambda i: (i, 0)),
        ],
        out_specs=pl.BlockSpec((bn, D), lambda i: (i, 0)),
        out_shape=jax.ShapeDtypeStruct((N, D), jnp.float32),
    )(p0, p1)


def kernel(x, edge_index, rnbrw_weights, W, a_src, a_dst):
    src = edge_index[0]
    dst = edge_index[1]
    rn2d = rnbrw_weights.reshape(E // RN_COLS, RN_COLS)
    h, s, b2d = _k1(x, W, a_src, a_dst, rn2d)
    s_flat = s.reshape(2 * N)
    bias = b2d.reshape(E)
    score, mxp = _k2(s_flat, src, dst, bias)
    ex, denp = _k3(mxp, dst, score)
    alpha, outp = _k4(denp, dst, src, ex, h)
    out = _k5(outp[0], outp[1])
    return out, alpha.reshape(E, 1)


# speedup vs baseline: 9.6205x; 1.4952x over previous
import functools

import jax
import jax.numpy as jnp
from jax import lax
from jax.experimental import pallas as pl
from jax.experimental.pallas import tpu as pltpu
from jax.experimental.pallas import tpu_sc as plsc

N = 10000
E = 320000
D = 128
EPS = 1e-08
NEG_INIT = -3.0e38

NC = 2
NS = 16
NW = NC * NS
EPW = E // NW
L = 16
CHUNK = 128

_MESH = plsc.VectorSubcoreMesh(core_axis_name="c", subcore_axis_name="s")


def _wid():
    return lax.axis_index("s") * NC + lax.axis_index("c")


def _k1_body(x_ref, w_ref, asrc_ref, adst_ref, rn_ref, h_ref, s_ref, b_ref):
    h = lax.dot_general(x_ref[...], w_ref[...],
                        dimension_numbers=(((1,), (1,)), ((), ())),
                        preferred_element_type=jnp.float32)
    h_ref[...] = h
    s_ref[:, 0:1] = jnp.sum(h * asrc_ref[...], axis=1, keepdims=True)
    s_ref[:, 1:2] = jnp.sum(h * adst_ref[...], axis=1, keepdims=True)
    b_ref[...] = jnp.log(rn_ref[...] + EPS)


RN_COLS = 2000


def _k1(x, w, a_src, a_dst, rn2d):
    grid = 10
    bn = N // grid
    br = (E // RN_COLS) // grid
    return pl.pallas_call(
        _k1_body,
        grid=(grid,),
        in_specs=[
            pl.BlockSpec((bn, D), lambda i: (i, 0)),
            pl.BlockSpec((D, D), lambda i: (0, 0)),
            pl.BlockSpec((1, D), lambda i: (0, 0)),
            pl.BlockSpec((1, D), lambda i: (0, 0)),
            pl.BlockSpec((br, RN_COLS), lambda i: (i, 0)),
        ],
        out_specs=[
            pl.BlockSpec((bn, D), lambda i: (i, 0)),
            pl.BlockSpec((bn, 2), lambda i: (i, 0)),
            pl.BlockSpec((br, RN_COLS), lambda i: (i, 0)),
        ],
        out_shape=[
            jax.ShapeDtypeStruct((N, D), jnp.float32),
            jax.ShapeDtypeStruct((N, 2), jnp.float32),
            jax.ShapeDtypeStruct((E // RN_COLS, RN_COLS), jnp.float32),
        ],
    )(x, w, a_src, a_dst, rn2d)


def _k2_body(s_hbm, src_hbm, dst_hbm, bias_hbm, score_hbm, mxp_hbm,
             s_v, src_v, dst_v, bias_v, score_v, mx_v):
    wid = _wid()
    base = wid * EPW
    pltpu.sync_copy(s_hbm, s_v)
    pltpu.sync_copy(src_hbm.at[pl.ds(base, EPW)], src_v)
    pltpu.sync_copy(dst_hbm.at[pl.ds(base, EPW)], dst_v)
    pltpu.sync_copy(bias_hbm.at[pl.ds(base, EPW)], bias_v)

    neg = jnp.full((L,), NEG_INIT, jnp.float32)

    def init_body(i, c):
        mx_v[pl.ds(i * L, L)] = neg
        return c

    lax.fori_loop(0, N // L, init_body, 0)

    def edge_body(i, c):
        sl = pl.ds(i * L, L)
        sv = src_v[sl]
        dv = dst_v[sl]
        se = plsc.load_gather(s_v, [sv * 2])
        de = plsc.load_gather(s_v, [dv * 2 + 1])
        sc = se + de + bias_v[sl]
        sc = jnp.where(sc >= 0.0, sc, 0.2 * sc)
        score_v[sl] = sc

        def cond(p):
            return jnp.any(p)

        def body(p):
            cur = plsc.load_gather(mx_v, [dv])
            plsc.store_scatter(mx_v, [dv], jnp.maximum(cur, sc), mask=p)
            chk = plsc.load_gather(mx_v, [dv])
            return jnp.logical_and(p, chk < sc)

        lax.while_loop(cond, body, jnp.ones((L,), jnp.bool_))
        return c

    lax.fori_loop(0, EPW // L, edge_body, 0)
    pltpu.sync_copy(score_v, score_hbm.at[pl.ds(base, EPW)])
    pltpu.sync_copy(mx_v, mxp_hbm.at[wid])


def _k2(s_flat, src, dst, bias):
    f = pl.kernel(
        _k2_body,
        out_type=[
            jax.ShapeDtypeStruct((E,), jnp.float32),
            jax.ShapeDtypeStruct((NW, N), jnp.float32),
        ],
        mesh=_MESH,
        compiler_params=pltpu.CompilerParams(needs_layout_passes=False),
        scratch_types=[
            pltpu.VMEM((2 * N,), jnp.float32),
            pltpu.VMEM((EPW,), jnp.int32),
            pltpu.VMEM((EPW,), jnp.int32),
            pltpu.VMEM((EPW,), jnp.float32),
            pltpu.VMEM((EPW,), jnp.float32),
            pltpu.VMEM((N,), jnp.float32),
        ],
    )
    return f(s_flat, src, dst, bias)


def _combine(part_hbm, acc_v, row_v, op):
    pltpu.sync_copy(part_hbm.at[0], acc_v)

    def row_body(r, c):
        pltpu.sync_copy(part_hbm.at[r], row_v)

        def vec_body(i, c2):
            sl = pl.ds(i * L, L)
            acc_v[sl] = op(acc_v[sl], row_v[sl])
            return c2

        lax.fori_loop(0, N // L, vec_body, 0)
        return c

    lax.fori_loop(1, NW, row_body, 0)


def _k3_body(mxp_hbm, dst_hbm, score_hbm, ex_hbm, denp_hbm,
             mx_v, row_v, dst_v, score_v, ex_v, den_v):
    wid = _wid()
    base = wid * EPW
    _combine(mxp_hbm, mx_v, row_v, jnp.maximum)
    pltpu.sync_copy(dst_hbm.at[pl.ds(base, EPW)], dst_v)
    pltpu.sync_copy(score_hbm.at[pl.ds(base, EPW)], score_v)

    zero = jnp.zeros((L,), jnp.float32)

    def init_body(i, c):
        den_v[pl.ds(i * L, L)] = zero
        return c

    lax.fori_loop(0, N // L, init_body, 0)

    def edge_body(i, c):
        sl = pl.ds(i * L, L)
        dv = dst_v[sl]
        ex = jnp.exp(score_v[sl] - plsc.load_gather(mx_v, [dv]))
        ex_v[sl] = ex
        plsc.addupdate_scatter(den_v, [dv], ex)
        return c

    lax.fori_loop(0, EPW // L, edge_body, 0)
    pltpu.sync_copy(ex_v, ex_hbm.at[pl.ds(base, EPW)])
    pltpu.sync_copy(den_v, denp_hbm.at[wid])


def _k3(mxp, dst, score):
    f = pl.kernel(
        _k3_body,
        out_type=[
            jax.ShapeDtypeStruct((E,), jnp.float32),
            jax.ShapeDtypeStruct((NW, N), jnp.float32),
        ],
        mesh=_MESH,
        compiler_params=pltpu.CompilerParams(needs_layout_passes=False),
        scratch_types=[
            pltpu.VMEM((N,), jnp.float32),
            pltpu.VMEM((N,), jnp.float32),
            pltpu.VMEM((EPW,), jnp.int32),
            pltpu.VMEM((EPW,), jnp.float32),
            pltpu.VMEM((EPW,), jnp.float32),
            pltpu.VMEM((N,), jnp.float32),
        ],
    )
    return f(mxp, dst, score)


T_BASES = (0, 3328, 6656)
T_SIZES = (3328, 3328, 3344)
ACC_ROWS = max(T_SIZES) + 8
ROWS_PER_TILE = 208


LCAP = EPW + CHUNK
MAXB = (EPW + CHUNK - 1) // CHUNK


def _k4_body(denp_hbm, dst_hbm, src_hbm, ex_hbm, h_hbm,
             alpha_hbm, outp_hbm,
             den_v, row_v, lst0_v, lst1_v, lst2_v, src_v, dst_v, ex_v,
             rows_v, sidx_v, didx_v, sem, acc):
    cid = lax.axis_index("c")
    sid = lax.axis_index("s")
    wid = sid * NC + cid
    base = wid * EPW
    pltpu.sync_copy(src_hbm.at[pl.ds(base, EPW)], src_v)
    pltpu.sync_copy(dst_hbm.at[pl.ds(base, EPW)], dst_v)
    pltpu.sync_copy(ex_hbm.at[pl.ds(base, EPW)], ex_v)
    _combine(denp_hbm, den_v, row_v, jnp.add)

    def alpha_body(i, c):
        sl = pl.ds(i * L, L)
        dv = dst_v[sl]
        ex_v[sl] = ex_v[sl] / (plsc.load_gather(den_v, [dv]) + EPS)
        return c

    lax.fori_loop(0, EPW // L, alpha_body, 0)
    pltpu.sync_copy(ex_v, alpha_hbm.at[pl.ds(base, EPW)])

    lane = lax.iota(jnp.int32, L)
    lists = (lst0_v, lst1_v, lst2_v)

    def build_body(i, cnts):
        c0, c1, c2 = cnts
        sl = pl.ds(i * L, L)
        dv = dst_v[sl]
        posf = plsc.bitcast(lane + i * L, jnp.float32)
        m0 = dv < T_BASES[1]
        m2 = dv >= T_BASES[2]
        m1 = jnp.logical_and(jnp.logical_not(m0), jnp.logical_not(m2))
        plsc.store_compressed(lst0_v.at[pl.ds(c0, L)], posf, mask=m0)
        plsc.store_compressed(lst1_v.at[pl.ds(c1, L)], posf, mask=m1)
        plsc.store_compressed(lst2_v.at[pl.ds(c2, L)], posf, mask=m2)
        one = jnp.int32(1)
        return (c0 + jnp.sum(jnp.where(m0, one, 0)),
                c1 + jnp.sum(jnp.where(m1, one, 0)),
                c2 + jnp.sum(jnp.where(m2, one, 0)))

    z = jnp.int32(0)
    cnts = lax.fori_loop(0, EPW // L, build_body, (z, z, z))

    zero = jnp.zeros((L,), jnp.float32)

    def zrow_body(j, c):
        for r in range(D // L):
            rows_v[j, pl.ds(r * L, L)] = zero
        return c

    lax.fori_loop(0, CHUNK, zrow_body, 0)

    for t, (tb, ts) in enumerate(zip(T_BASES, T_SIZES)):
        lst = lists[t]
        cnt = cnts[t]
        for zoff in range(0, ROWS_PER_TILE, CHUNK):
            zsz = min(CHUNK, ROWS_PER_TILE - zoff)
            pltpu.sync_copy(
                rows_v.at[pl.ds(0, zsz)],
                acc.at[pl.ds(sid * ROWS_PER_TILE + zoff, zsz)])

        @pl.when(sid == 0)
        def _zero_tail():
            extra = ts + 8 - NS * ROWS_PER_TILE
            pltpu.sync_copy(rows_v.at[pl.ds(0, extra)],
                            acc.at[pl.ds(NS * ROWS_PER_TILE, extra)])

        plsc.subcore_barrier()

        def batch_body(b, carry):
            off = b * CHUNK

            def ib(k, c2):
                koff = off + k * L
                pos = plsc.bitcast(lst[pl.ds(koff, L)], jnp.int32)
                valid = (koff + lane) < cnt
                pos0 = jnp.where(valid, pos, 0)
                sidx_v[pl.ds(k * L, L)] = plsc.load_gather(src_v, [pos0])
                dvv = plsc.load_gather(dst_v, [pos0])
                didx_v[pl.ds(k * L, L)] = jnp.where(
                    valid, dvv - tb, ts + (lane & 7))
                return c2

            lax.fori_loop(0, CHUNK // L, ib, 0)
            gather = pltpu.async_copy(h_hbm.at[sidx_v], rows_v, sem)
            gather.wait()

            def scale_body(j, c2):
                jj = jnp.minimum(jnp.zeros((L,), jnp.int32) + (off + j),
                                 cnt - 1)
                pj = plsc.bitcast(plsc.load_gather(lst, [jj]), jnp.int32)
                a_b = plsc.load_gather(ex_v, [pj])
                for r in range(D // L):
                    sl = pl.ds(r * L, L)
                    rows_v[j, sl] = rows_v[j, sl] * a_b
                return c2

            lax.fori_loop(0, CHUNK, scale_body, 0)
            pltpu.sync_copy(rows_v, acc.at[didx_v], add=True)
            return carry

        nb = lax.shift_right_logical(cnt + (CHUNK - 1), 7)
        lax.fori_loop(0, nb, batch_body, 0)
        plsc.subcore_barrier()

        pltpu.sync_copy(
            acc.at[pl.ds(sid * ROWS_PER_TILE, ROWS_PER_TILE)],
            outp_hbm.at[cid].at[pl.ds(tb + sid * ROWS_PER_TILE,
                                      ROWS_PER_TILE)])

        if ts > NS * ROWS_PER_TILE:
            @pl.when(sid == 0)
            def _dump_tail():
                extra = ts - NS * ROWS_PER_TILE
                pltpu.sync_copy(
                    acc.at[pl.ds(NS * ROWS_PER_TILE, extra)],
                    outp_hbm.at[cid].at[pl.ds(tb + NS * ROWS_PER_TILE,
                                              extra)])

        plsc.subcore_barrier()

        if t < 2:
            lax.fori_loop(0, CHUNK, zrow_body, 0)


def _k4(denp, dst, src, ex, h):
    f = pl.kernel(
        _k4_body,
        out_type=[
            jax.ShapeDtypeStruct((E,), jnp.float32),
            jax.ShapeDtypeStruct((NC, N, D), jnp.float32),
        ],
        mesh=_MESH,
        compiler_params=pltpu.CompilerParams(needs_layout_passes=False),
        scratch_types=[
            pltpu.VMEM((N,), jnp.float32),
            pltpu.VMEM((N,), jnp.float32),
            pltpu.VMEM((LCAP,), jnp.float32),
            pltpu.VMEM((LCAP,), jnp.float32),
            pltpu.VMEM((LCAP,), jnp.float32),
            pltpu.VMEM((EPW,), jnp.int32),
            pltpu.VMEM((EPW,), jnp.int32),
            pltpu.VMEM((EPW,), jnp.float32),
            pltpu.VMEM((CHUNK, D), jnp.float32),
            pltpu.VMEM((CHUNK,), jnp.int32),
            pltpu.VMEM((CHUNK,), jnp.int32),
            pltpu.SemaphoreType.DMA,
            pltpu.VMEM_SHARED((ACC_ROWS, D), jnp.float32),
        ],
    )
    return f(denp, dst, src, ex, h)


def _k5_body(a_ref, b_ref, o_ref):
    o_ref[...] = a_ref[...] + b_ref[...]


def _k5(p0, p1):
    grid = 10
    bn = N // grid
    return pl.pallas_call(
        _k5_body,
        grid=(grid,),
        in_specs=[
            pl.BlockSpec((bn, D), lambda i: (i, 0)),
            pl.BlockSpec((bn, D), lambda i: (i, 0)),
        ],
        out_specs=pl.BlockSpec((bn, D), lambda i: (i, 0)),
        out_shape=jax.ShapeDtypeStruct((N, D), jnp.float32),
    )(p0, p1)


def kernel(x, edge_index, rnbrw_weights, W, a_src, a_dst):
    src = edge_index[0]
    dst = edge_index[1]
    rn2d = rnbrw_weights.reshape(E // RN_COLS, RN_COLS)
    h, s, b2d = _k1(x, W, a_src, a_dst, rn2d)
    s_flat = s.reshape(2 * N)
    bias = b2d.reshape(E)
    score, mxp = _k2(s_flat, src, dst, bias)
    ex, denp = _k3(mxp, dst, score)
    alpha, outp = _k4(denp, dst, src, ex, h)
    out = _k5(outp[0], outp[1])
    return out, alpha.reshape(E, 1)


# speedup vs baseline: 13.6325x; 1.4170x over previous
import functools

import jax
import jax.numpy as jnp
from jax import lax
from jax.experimental import pallas as pl
from jax.experimental.pallas import tpu as pltpu
from jax.experimental.pallas import tpu_sc as plsc

N = 10000
E = 320000
D = 128
EPS = 1e-08
NEG_INIT = -3.0e38

NC = 2
NS = 16
NW = NC * NS
EPW = E // NW
L = 16
NP = 10240
SLICE = NP // NS
CHUNK = 128

_MESH = plsc.VectorSubcoreMesh(core_axis_name="c", subcore_axis_name="s")


def _wid():
    return lax.axis_index("s") * NC + lax.axis_index("c")


def _k1_body(x_ref, w_ref, asrc_ref, adst_ref, rn_ref, h_ref, s_ref, b_ref):
    h = lax.dot_general(x_ref[...], w_ref[...],
                        dimension_numbers=(((1,), (1,)), ((), ())),
                        preferred_element_type=jnp.float32)
    h_ref[...] = h
    s_ref[:, 0:1] = jnp.sum(h * asrc_ref[...], axis=1, keepdims=True)
    s_ref[:, 1:2] = jnp.sum(h * adst_ref[...], axis=1, keepdims=True)
    b_ref[...] = jnp.log(rn_ref[...] + EPS)


RN_COLS = 2000


def _k1(x, w, a_src, a_dst, rn2d):
    grid = 10
    bn = N // grid
    br = (E // RN_COLS) // grid
    return pl.pallas_call(
        _k1_body,
        grid=(grid,),
        in_specs=[
            pl.BlockSpec((bn, D), lambda i: (i, 0)),
            pl.BlockSpec((D, D), lambda i: (0, 0)),
            pl.BlockSpec((1, D), lambda i: (0, 0)),
            pl.BlockSpec((1, D), lambda i: (0, 0)),
            pl.BlockSpec((br, RN_COLS), lambda i: (i, 0)),
        ],
        out_specs=[
            pl.BlockSpec((bn, D), lambda i: (i, 0)),
            pl.BlockSpec((bn, 2), lambda i: (i, 0)),
            pl.BlockSpec((br, RN_COLS), lambda i: (i, 0)),
        ],
        out_shape=[
            jax.ShapeDtypeStruct((N, D), jnp.float32),
            jax.ShapeDtypeStruct((N, 2), jnp.float32),
            jax.ShapeDtypeStruct((E // RN_COLS, RN_COLS), jnp.float32),
        ],
    )(x, w, a_src, a_dst, rn2d)


def _k2_body(s_hbm, src_hbm, dst_hbm, bias_hbm, score_hbm, mxp_hbm,
             s_v, src_v, dst_v, bias_v, score_v, mx_v):
    wid = _wid()
    base = wid * EPW
    pltpu.sync_copy(s_hbm, s_v)
    pltpu.sync_copy(src_hbm.at[pl.ds(base, EPW)], src_v)
    pltpu.sync_copy(dst_hbm.at[pl.ds(base, EPW)], dst_v)
    pltpu.sync_copy(bias_hbm.at[pl.ds(base, EPW)], bias_v)

    neg = jnp.full((L,), NEG_INIT, jnp.float32)

    def init_body(i, c):
        mx_v[pl.ds(i * L, L)] = neg
        return c

    lax.fori_loop(0, NP // L, init_body, 0)

    def edge_body(i, c):
        sl = pl.ds(i * L, L)
        sv = src_v[sl]
        dv = dst_v[sl]
        se = plsc.load_gather(s_v, [sv * 2])
        de = plsc.load_gather(s_v, [dv * 2 + 1])
        sc = se + de + bias_v[sl]
        sc = jnp.where(sc >= 0.0, sc, 0.2 * sc)
        score_v[sl] = sc

        def cond(p):
            return jnp.any(p)

        def body(p):
            cur = plsc.load_gather(mx_v, [dv])
            plsc.store_scatter(mx_v, [dv], jnp.maximum(cur, sc), mask=p)
            chk = plsc.load_gather(mx_v, [dv])
            return jnp.logical_and(p, chk < sc)

        lax.while_loop(cond, body, jnp.ones((L,), jnp.bool_))
        return c

    lax.fori_loop(0, EPW // L, edge_body, 0)
    pltpu.sync_copy(score_v, score_hbm.at[pl.ds(base, EPW)])
    pltpu.sync_copy(mx_v, mxp_hbm.at[wid])


def _k2(s_flat, src, dst, bias):
    f = pl.kernel(
        _k2_body,
        out_type=[
            jax.ShapeDtypeStruct((E,), jnp.float32),
            jax.ShapeDtypeStruct((NW, NP), jnp.float32),
        ],
        mesh=_MESH,
        compiler_params=pltpu.CompilerParams(needs_layout_passes=False),
        scratch_types=[
            pltpu.VMEM((2 * N,), jnp.float32),
            pltpu.VMEM((EPW,), jnp.int32),
            pltpu.VMEM((EPW,), jnp.int32),
            pltpu.VMEM((EPW,), jnp.float32),
            pltpu.VMEM((EPW,), jnp.float32),
            pltpu.VMEM((NP,), jnp.float32),
        ],
    )
    return f(s_flat, src, dst, bias)


def _combine(part_hbm, full_v, buf_v, red_v, sp, sem, op):
    sid = lax.axis_index("s")
    off = sid * SLICE
    descs = [pltpu.async_copy(part_hbm.at[r].at[pl.ds(off, SLICE)],
                              buf_v.at[r], sem)
             for r in range(NW)]
    for d in descs:
        d.wait()

    def vec_body(i, c):
        sl = pl.ds(i * L, L)

        def row_body(r, a):
            return op(a, buf_v[r, sl])

        red_v[sl] = lax.fori_loop(1, NW, row_body, buf_v[0, sl])
        return c

    lax.fori_loop(0, SLICE // L, vec_body, 0)
    pltpu.sync_copy(red_v, sp.at[pl.ds(off, SLICE)])
    plsc.subcore_barrier()
    pltpu.sync_copy(sp, full_v)


NR = NP // L


def _k3_body(mxp_hbm, dst_hbm, score_hbm, ex_hbm, denp_hbm,
             mx_v, buf_v, red_v, dst_v, score_v, ex_v, den_v, sem, sp):
    wid = _wid()
    base = wid * EPW
    _combine(mxp_hbm, mx_v, buf_v, red_v, sp, sem, jnp.maximum)
    pltpu.sync_copy(dst_hbm.at[pl.ds(base, EPW)], dst_v)
    pltpu.sync_copy(score_hbm.at[pl.ds(base, EPW)], score_v)

    zero = jnp.zeros((L,), jnp.float32)

    def init_body(i, c):
        den_v[pl.ds(i * L, L)] = zero
        return c

    lax.fori_loop(0, NP // L, init_body, 0)

    def edge_body(i, c):
        sl = pl.ds(i * L, L)
        dv = dst_v[sl]
        ex = jnp.exp(score_v[sl] - plsc.load_gather(mx_v, [dv]))
        ex_v[sl] = ex
        plsc.addupdate_scatter(den_v, [dv], ex)
        return c

    lax.fori_loop(0, EPW // L, edge_body, 0)
    pltpu.sync_copy(ex_v, ex_hbm.at[pl.ds(base, EPW)])
    pltpu.sync_copy(den_v, denp_hbm.at[wid])


def _k3(mxp, dst, score):
    f = pl.kernel(
        _k3_body,
        out_type=[
            jax.ShapeDtypeStruct((E,), jnp.float32),
            jax.ShapeDtypeStruct((NW, NP), jnp.float32),
        ],
        mesh=_MESH,
        compiler_params=pltpu.CompilerParams(needs_layout_passes=False),
        scratch_types=[
            pltpu.VMEM((NP,), jnp.float32),
            pltpu.VMEM((NW, SLICE), jnp.float32),
            pltpu.VMEM((SLICE,), jnp.float32),
            pltpu.VMEM((EPW,), jnp.int32),
            pltpu.VMEM((EPW,), jnp.float32),
            pltpu.VMEM((EPW,), jnp.float32),
            pltpu.VMEM((NP,), jnp.float32),
            pltpu.SemaphoreType.DMA,
            pltpu.VMEM_SHARED((NP,), jnp.float32),
        ],
    )
    return f(mxp, dst, score)


def _k3b_body(denp_hbm, denc_hbm, buf_v, red_v, sem):
    cid = lax.axis_index("c")
    sid = lax.axis_index("s")

    @pl.when(cid == 0)
    def _do():
        off = sid * SLICE
        descs = [pltpu.async_copy(denp_hbm.at[r].at[pl.ds(off, SLICE)],
                                  buf_v.at[r], sem)
                 for r in range(NW)]
        for d in descs:
            d.wait()

        def vec_body(i, c):
            sl = pl.ds(i * L, L)

            def row_body(r, a):
                return a + buf_v[r, sl]

            red_v[sl] = lax.fori_loop(1, NW, row_body, buf_v[0, sl])
            return c

        lax.fori_loop(0, SLICE // L, vec_body, 0)
        pltpu.sync_copy(red_v, denc_hbm.at[pl.ds(off, SLICE)])


def _k3b(denp):
    f = pl.kernel(
        _k3b_body,
        out_type=jax.ShapeDtypeStruct((NP,), jnp.float32),
        mesh=_MESH,
        compiler_params=pltpu.CompilerParams(needs_layout_passes=False),
        scratch_types=[
            pltpu.VMEM((NW, SLICE), jnp.float32),
            pltpu.VMEM((SLICE,), jnp.float32),
            pltpu.SemaphoreType.DMA,
        ],
    )
    return f(denp)


T_BASES = (0, 3328, 6656)
T_SIZES = (3328, 3328, 3344)
ACC_ROWS = max(T_SIZES) + 8
ROWS_PER_TILE = 208
LCAP = EPW + CHUNK
MAXB = (EPW + CHUNK - 1) // CHUNK


def _k4_body(denc_hbm, dst_hbm, src_hbm, ex_hbm, h_hbm,
             alpha_hbm, outp_hbm,
             den_v, lst0_v, lst1_v, lst2_v, src_v, dst_v,
             ex_v, rows_v, sidx_v, didx_v, sem, acc):
    cid = lax.axis_index("c")
    sid = lax.axis_index("s")
    wid = sid * NC + cid
    base = wid * EPW
    pltpu.sync_copy(src_hbm.at[pl.ds(base, EPW)], src_v)
    pltpu.sync_copy(dst_hbm.at[pl.ds(base, EPW)], dst_v)
    pltpu.sync_copy(ex_hbm.at[pl.ds(base, EPW)], ex_v)
    pltpu.sync_copy(denc_hbm, den_v)

    def alpha_body(i, c):
        sl = pl.ds(i * L, L)
        dv = dst_v[sl]
        ex_v[sl] = ex_v[sl] / (plsc.load_gather(den_v, [dv]) + EPS)
        return c

    lax.fori_loop(0, EPW // L, alpha_body, 0)
    pltpu.sync_copy(ex_v, alpha_hbm.at[pl.ds(base, EPW)])

    lane = lax.iota(jnp.int32, L)
    lists = (lst0_v, lst1_v, lst2_v)

    def build_body(i, cnts):
        c0, c1, c2 = cnts
        sl = pl.ds(i * L, L)
        dv = dst_v[sl]
        posf = plsc.bitcast(lane + i * L, jnp.float32)
        m0 = dv < T_BASES[1]
        m2 = dv >= T_BASES[2]
        m1 = jnp.logical_and(jnp.logical_not(m0), jnp.logical_not(m2))
        plsc.store_compressed(lst0_v.at[pl.ds(c0, L)], posf, mask=m0)
        plsc.store_compressed(lst1_v.at[pl.ds(c1, L)], posf, mask=m1)
        plsc.store_compressed(lst2_v.at[pl.ds(c2, L)], posf, mask=m2)
        one = jnp.int32(1)
        return (c0 + jnp.sum(jnp.where(m0, one, 0)),
                c1 + jnp.sum(jnp.where(m1, one, 0)),
                c2 + jnp.sum(jnp.where(m2, one, 0)))

    z = jnp.int32(0)
    cnts = lax.fori_loop(0, EPW // L, build_body, (z, z, z))

    zero = jnp.zeros((L,), jnp.float32)

    def zrow_body(j, c):
        for r in range(D // L):
            rows_v[j, pl.ds(r * L, L)] = zero
        return c

    lax.fori_loop(0, CHUNK, zrow_body, 0)

    for t, (tb, ts) in enumerate(zip(T_BASES, T_SIZES)):
        lst = lists[t]
        cnt = cnts[t]
        for zoff in range(0, ROWS_PER_TILE, CHUNK):
            zsz = min(CHUNK, ROWS_PER_TILE - zoff)
            pltpu.sync_copy(
                rows_v.at[pl.ds(0, zsz)],
                acc.at[pl.ds(sid * ROWS_PER_TILE + zoff, zsz)])

        @pl.when(sid == 0)
        def _zero_tail():
            extra = ts + 8 - NS * ROWS_PER_TILE
            pltpu.sync_copy(rows_v.at[pl.ds(0, extra)],
                            acc.at[pl.ds(NS * ROWS_PER_TILE, extra)])

        plsc.subcore_barrier()

        def batch_body(b, carry):
            off = b * CHUNK

            def ib(k, c2):
                koff = off + k * L
                pos = plsc.bitcast(lst[pl.ds(koff, L)], jnp.int32)
                valid = (koff + lane) < cnt
                pos0 = jnp.where(valid, pos, 0)
                sidx_v[pl.ds(k * L, L)] = plsc.load_gather(src_v, [pos0])
                dvv = plsc.load_gather(dst_v, [pos0])
                didx_v[pl.ds(k * L, L)] = jnp.where(
                    valid, dvv - tb, ts + (lane & 7))
                return c2

            lax.fori_loop(0, CHUNK // L, ib, 0)
            gather = pltpu.async_copy(h_hbm.at[sidx_v], rows_v, sem)
            gather.wait()

            def scale_body(j, c2):
                jj = jnp.minimum(jnp.zeros((L,), jnp.int32) + (off + j),
                                 cnt - 1)
                pj = plsc.bitcast(plsc.load_gather(lst, [jj]), jnp.int32)
                a_b = plsc.load_gather(ex_v, [pj])
                for r in range(D // L):
                    sl = pl.ds(r * L, L)
                    rows_v[j, sl] = rows_v[j, sl] * a_b
                return c2

            lax.fori_loop(0, CHUNK, scale_body, 0)
            pltpu.sync_copy(rows_v, acc.at[didx_v], add=True)
            return carry

        nb = lax.shift_right_logical(cnt + (CHUNK - 1), 7)
        lax.fori_loop(0, nb, batch_body, 0)
        plsc.subcore_barrier()

        pltpu.sync_copy(
            acc.at[pl.ds(sid * ROWS_PER_TILE, ROWS_PER_TILE)],
            outp_hbm.at[cid].at[pl.ds(tb + sid * ROWS_PER_TILE,
                                      ROWS_PER_TILE)])

        if ts > NS * ROWS_PER_TILE:
            @pl.when(sid == 0)
            def _dump_tail():
                extra = ts - NS * ROWS_PER_TILE
                pltpu.sync_copy(
                    acc.at[pl.ds(NS * ROWS_PER_TILE, extra)],
                    outp_hbm.at[cid].at[pl.ds(tb + NS * ROWS_PER_TILE,
                                              extra)])

        plsc.subcore_barrier()

        if t < 2:
            lax.fori_loop(0, CHUNK, zrow_body, 0)


def _k4(denc, dst, src, ex, h):
    f = pl.kernel(
        _k4_body,
        out_type=[
            jax.ShapeDtypeStruct((E,), jnp.float32),
            jax.ShapeDtypeStruct((NC, N, D), jnp.float32),
        ],
        mesh=_MESH,
        compiler_params=pltpu.CompilerParams(needs_layout_passes=False),
        scratch_types=[
            pltpu.VMEM((NP,), jnp.float32),
            pltpu.VMEM((LCAP,), jnp.float32),
            pltpu.VMEM((LCAP,), jnp.float32),
            pltpu.VMEM((LCAP,), jnp.float32),
            pltpu.VMEM((EPW,), jnp.int32),
            pltpu.VMEM((EPW,), jnp.int32),
            pltpu.VMEM((EPW,), jnp.float32),
            pltpu.VMEM((CHUNK, D), jnp.float32),
            pltpu.VMEM((CHUNK,), jnp.int32),
            pltpu.VMEM((CHUNK,), jnp.int32),
            pltpu.SemaphoreType.DMA,
            pltpu.VMEM_SHARED((ACC_ROWS, D), jnp.float32),
        ],
    )
    return f(denc, dst, src, ex, h)


def _k5_body(a_ref, b_ref, o_ref):
    o_ref[...] = a_ref[...] + b_ref[...]


def _k5(p0, p1):
    grid = 10
    bn = N // grid
    return pl.pallas_call(
        _k5_body,
        grid=(grid,),
        in_specs=[
            pl.BlockSpec((bn, D), lambda i: (i, 0)),
            pl.BlockSpec((bn, D), lambda i: (i, 0)),
        ],
        out_specs=pl.BlockSpec((bn, D), lambda i: (i, 0)),
        out_shape=jax.ShapeDtypeStruct((N, D), jnp.float32),
    )(p0, p1)


def kernel(x, edge_index, rnbrw_weights, W, a_src, a_dst):
    src = edge_index[0]
    dst = edge_index[1]
    rn2d = rnbrw_weights.reshape(E // RN_COLS, RN_COLS)
    h, s, b2d = _k1(x, W, a_src, a_dst, rn2d)
    s_flat = s.reshape(2 * N)
    bias = b2d.reshape(E)
    score, mxp = _k2(s_flat, src, dst, bias)
    ex, denp = _k3(mxp, dst, score)
    denc = _k3b(denp)
    alpha, outp = _k4(denc, dst, src, ex, h)
    out = _k5(outp[0], outp[1])
    return out, alpha.reshape(E, 1)


# speedup vs baseline: 14.8250x; 1.0875x over previous
import functools

import jax
import jax.numpy as jnp
from jax import lax
from jax.experimental import pallas as pl
from jax.experimental.pallas import tpu as pltpu
from jax.experimental.pallas import tpu_sc as plsc

N = 10000
E = 320000
D = 128
EPS = 1e-08
NEG_INIT = -3.0e38

NC = 2
NS = 16
NW = NC * NS
EPW = E // NW
L = 16
NP = 10240
SLICE = NP // NS
CHUNK = 64

_MESH = plsc.VectorSubcoreMesh(core_axis_name="c", subcore_axis_name="s")


def _wid():
    return lax.axis_index("s") * NC + lax.axis_index("c")


def _k1_body(x_ref, w_ref, asrc_ref, adst_ref, rn_ref, h_ref, s_ref, b_ref):
    h = lax.dot_general(x_ref[...], w_ref[...],
                        dimension_numbers=(((1,), (1,)), ((), ())),
                        preferred_element_type=jnp.float32)
    h_ref[...] = h
    s_ref[:, 0:1] = jnp.sum(h * asrc_ref[...], axis=1, keepdims=True)
    s_ref[:, 1:2] = jnp.sum(h * adst_ref[...], axis=1, keepdims=True)
    b_ref[...] = jnp.log(rn_ref[...] + EPS)


RN_COLS = 2000


def _k1(x, w, a_src, a_dst, rn2d):
    grid = 10
    bn = N // grid
    br = (E // RN_COLS) // grid
    return pl.pallas_call(
        _k1_body,
        grid=(grid,),
        in_specs=[
            pl.BlockSpec((bn, D), lambda i: (i, 0)),
            pl.BlockSpec((D, D), lambda i: (0, 0)),
            pl.BlockSpec((1, D), lambda i: (0, 0)),
            pl.BlockSpec((1, D), lambda i: (0, 0)),
            pl.BlockSpec((br, RN_COLS), lambda i: (i, 0)),
        ],
        out_specs=[
            pl.BlockSpec((bn, D), lambda i: (i, 0)),
            pl.BlockSpec((bn, 2), lambda i: (i, 0)),
            pl.BlockSpec((br, RN_COLS), lambda i: (i, 0)),
        ],
        out_shape=[
            jax.ShapeDtypeStruct((N, D), jnp.float32),
            jax.ShapeDtypeStruct((N, 2), jnp.float32),
            jax.ShapeDtypeStruct((E // RN_COLS, RN_COLS), jnp.float32),
        ],
    )(x, w, a_src, a_dst, rn2d)


def _k2_body(s_hbm, src_hbm, dst_hbm, bias_hbm, score_hbm, mxp_hbm,
             s_v, src_v, dst_v, bias_v, score_v, mx_v):
    wid = _wid()
    base = wid * EPW
    pltpu.sync_copy(s_hbm, s_v)
    pltpu.sync_copy(src_hbm.at[pl.ds(base, EPW)], src_v)
    pltpu.sync_copy(dst_hbm.at[pl.ds(base, EPW)], dst_v)
    pltpu.sync_copy(bias_hbm.at[pl.ds(base, EPW)], bias_v)

    neg = jnp.full((L,), NEG_INIT, jnp.float32)

    def init_body(i, c):
        mx_v[pl.ds(i * L, L)] = neg
        return c

    lax.fori_loop(0, NP // L, init_body, 0)

    def edge_body(i, c):
        sl = pl.ds(i * L, L)
        sv = src_v[sl]
        dv = dst_v[sl]
        se = plsc.load_gather(s_v, [sv * 2])
        de = plsc.load_gather(s_v, [dv * 2 + 1])
        sc = se + de + bias_v[sl]
        sc = jnp.where(sc >= 0.0, sc, 0.2 * sc)
        score_v[sl] = sc

        def cond(p):
            return jnp.any(p)

        def body(p):
            cur = plsc.load_gather(mx_v, [dv])
            plsc.store_scatter(mx_v, [dv], jnp.maximum(cur, sc), mask=p)
            chk = plsc.load_gather(mx_v, [dv])
            return jnp.logical_and(p, chk < sc)

        lax.while_loop(cond, body, jnp.ones((L,), jnp.bool_))
        return c

    lax.fori_loop(0, EPW // L, edge_body, 0)
    pltpu.sync_copy(score_v, score_hbm.at[pl.ds(base, EPW)])
    pltpu.sync_copy(mx_v, mxp_hbm.at[wid])


def _k2(s_flat, src, dst, bias):
    f = pl.kernel(
        _k2_body,
        out_type=[
            jax.ShapeDtypeStruct((E,), jnp.float32),
            jax.ShapeDtypeStruct((NW, NP), jnp.float32),
        ],
        mesh=_MESH,
        compiler_params=pltpu.CompilerParams(needs_layout_passes=False),
        scratch_types=[
            pltpu.VMEM((2 * N,), jnp.float32),
            pltpu.VMEM((EPW,), jnp.int32),
            pltpu.VMEM((EPW,), jnp.int32),
            pltpu.VMEM((EPW,), jnp.float32),
            pltpu.VMEM((EPW,), jnp.float32),
            pltpu.VMEM((NP,), jnp.float32),
        ],
    )
    return f(s_flat, src, dst, bias)


def _combine(part_hbm, full_v, buf_v, red_v, sp, sem, op):
    sid = lax.axis_index("s")
    off = sid * SLICE
    descs = [pltpu.async_copy(part_hbm.at[r].at[pl.ds(off, SLICE)],
                              buf_v.at[r], sem)
             for r in range(NW)]
    for d in descs:
        d.wait()

    def vec_body(i, c):
        sl = pl.ds(i * L, L)

        def row_body(r, a):
            return op(a, buf_v[r, sl])

        red_v[sl] = lax.fori_loop(1, NW, row_body, buf_v[0, sl])
        return c

    lax.fori_loop(0, SLICE // L, vec_body, 0)
    pltpu.sync_copy(red_v, sp.at[pl.ds(off, SLICE)])
    plsc.subcore_barrier()
    pltpu.sync_copy(sp, full_v)


NR = NP // L


def _k3_body(mxp_hbm, dst_hbm, score_hbm, ex_hbm, denp_hbm,
             mx_v, buf_v, red_v, dst_v, score_v, ex_v, den_v, sem, sp):
    wid = _wid()
    base = wid * EPW
    _combine(mxp_hbm, mx_v, buf_v, red_v, sp, sem, jnp.maximum)
    pltpu.sync_copy(dst_hbm.at[pl.ds(base, EPW)], dst_v)
    pltpu.sync_copy(score_hbm.at[pl.ds(base, EPW)], score_v)

    zero = jnp.zeros((L,), jnp.float32)

    def init_body(i, c):
        den_v[pl.ds(i * L, L)] = zero
        return c

    lax.fori_loop(0, NP // L, init_body, 0)

    def edge_body(i, c):
        sl = pl.ds(i * L, L)
        dv = dst_v[sl]
        ex = jnp.exp(score_v[sl] - plsc.load_gather(mx_v, [dv]))
        ex_v[sl] = ex
        plsc.addupdate_scatter(den_v, [dv], ex)
        return c

    lax.fori_loop(0, EPW // L, edge_body, 0)
    pltpu.sync_copy(ex_v, ex_hbm.at[pl.ds(base, EPW)])
    pltpu.sync_copy(den_v, denp_hbm.at[wid])


def _k3(mxp, dst, score):
    f = pl.kernel(
        _k3_body,
        out_type=[
            jax.ShapeDtypeStruct((E,), jnp.float32),
            jax.ShapeDtypeStruct((NW, NP), jnp.float32),
        ],
        mesh=_MESH,
        compiler_params=pltpu.CompilerParams(needs_layout_passes=False),
        scratch_types=[
            pltpu.VMEM((NP,), jnp.float32),
            pltpu.VMEM((NW, SLICE), jnp.float32),
            pltpu.VMEM((SLICE,), jnp.float32),
            pltpu.VMEM((EPW,), jnp.int32),
            pltpu.VMEM((EPW,), jnp.float32),
            pltpu.VMEM((EPW,), jnp.float32),
            pltpu.VMEM((NP,), jnp.float32),
            pltpu.SemaphoreType.DMA,
            pltpu.VMEM_SHARED((NP,), jnp.float32),
        ],
    )
    return f(mxp, dst, score)


def _k3b_body(denp_hbm, denc_hbm, buf_v, red_v, sem):
    cid = lax.axis_index("c")
    sid = lax.axis_index("s")

    @pl.when(cid == 0)
    def _do():
        off = sid * SLICE
        descs = [pltpu.async_copy(denp_hbm.at[r].at[pl.ds(off, SLICE)],
                                  buf_v.at[r], sem)
                 for r in range(NW)]
        for d in descs:
            d.wait()

        def vec_body(i, c):
            sl = pl.ds(i * L, L)

            def row_body(r, a):
                return a + buf_v[r, sl]

            red_v[sl] = lax.fori_loop(1, NW, row_body, buf_v[0, sl])
            return c

        lax.fori_loop(0, SLICE // L, vec_body, 0)
        pltpu.sync_copy(red_v, denc_hbm.at[pl.ds(off, SLICE)])


def _k3b(denp):
    f = pl.kernel(
        _k3b_body,
        out_type=jax.ShapeDtypeStruct((NP,), jnp.float32),
        mesh=_MESH,
        compiler_params=pltpu.CompilerParams(needs_layout_passes=False),
        scratch_types=[
            pltpu.VMEM((NW, SLICE), jnp.float32),
            pltpu.VMEM((SLICE,), jnp.float32),
            pltpu.SemaphoreType.DMA,
        ],
    )
    return f(denp)


T_BASES = (0, 3328, 6656)
T_SIZES = (3328, 3328, 3344)
ACC_ROWS = max(T_SIZES) + 8
ROWS_PER_TILE = 208
LCAP = EPW + 4 * CHUNK
MAXB = (EPW + CHUNK - 1) // CHUNK


def _k4_body(denc_hbm, dst_hbm, src_hbm, ex_hbm, h_hbm,
             alpha_hbm, outp_hbm,
             den_v, lst0_v, lst1_v, lst2_v, src_v, dst_v, ex_v,
             rows_a, rows_b, sidx_a, sidx_b, didx_a, didx_b,
             sga, sgb, ssa, ssb, acc):
    cid = lax.axis_index("c")
    sid = lax.axis_index("s")
    wid = sid * NC + cid
    base = wid * EPW
    pltpu.sync_copy(src_hbm.at[pl.ds(base, EPW)], src_v)
    pltpu.sync_copy(dst_hbm.at[pl.ds(base, EPW)], dst_v)
    pltpu.sync_copy(ex_hbm.at[pl.ds(base, EPW)], ex_v)
    pltpu.sync_copy(denc_hbm, den_v)

    def alpha_body(i, c):
        sl = pl.ds(i * L, L)
        dv = dst_v[sl]
        ex_v[sl] = ex_v[sl] / (plsc.load_gather(den_v, [dv]) + EPS)
        return c

    lax.fori_loop(0, EPW // L, alpha_body, 0)
    pltpu.sync_copy(ex_v, alpha_hbm.at[pl.ds(base, EPW)])

    lane = lax.iota(jnp.int32, L)
    lists = (lst0_v, lst1_v, lst2_v)

    def build_body(i, cnts):
        c0, c1, c2 = cnts
        sl = pl.ds(i * L, L)
        dv = dst_v[sl]
        posf = plsc.bitcast(lane + i * L, jnp.float32)
        m0 = dv < T_BASES[1]
        m2 = dv >= T_BASES[2]
        m1 = jnp.logical_and(jnp.logical_not(m0), jnp.logical_not(m2))
        plsc.store_compressed(lst0_v.at[pl.ds(c0, L)], posf, mask=m0)
        plsc.store_compressed(lst1_v.at[pl.ds(c1, L)], posf, mask=m1)
        plsc.store_compressed(lst2_v.at[pl.ds(c2, L)], posf, mask=m2)
        one = jnp.int32(1)
        return (c0 + jnp.sum(jnp.where(m0, one, 0)),
                c1 + jnp.sum(jnp.where(m1, one, 0)),
                c2 + jnp.sum(jnp.where(m2, one, 0)))

    z = jnp.int32(0)
    cnts = lax.fori_loop(0, EPW // L, build_body, (z, z, z))

    zero = jnp.zeros((L,), jnp.float32)

    def zrow_body(j, c):
        for r in range(D // L):
            rows_a[j, pl.ds(r * L, L)] = zero
        return c

    lax.fori_loop(0, CHUNK, zrow_body, 0)

    for t, (tb, ts) in enumerate(zip(T_BASES, T_SIZES)):
        lst = lists[t]
        cnt = cnts[t]
        for zoff in range(0, ROWS_PER_TILE, CHUNK):
            zsz = min(CHUNK, ROWS_PER_TILE - zoff)
            pltpu.sync_copy(
                rows_a.at[pl.ds(0, zsz)],
                acc.at[pl.ds(sid * ROWS_PER_TILE + zoff, zsz)])

        @pl.when(sid == 0)
        def _zero_tail():
            extra = ts + 8 - NS * ROWS_PER_TILE
            pltpu.sync_copy(rows_a.at[pl.ds(0, extra)],
                            acc.at[pl.ds(NS * ROWS_PER_TILE, extra)])

        plsc.subcore_barrier()

        def build_idx(off, sidx_ref, didx_ref):
            def ib(k, c2):
                koff = off + k * L
                pos = plsc.bitcast(lst[pl.ds(koff, L)], jnp.int32)
                valid = (koff + lane) < cnt
                pos0 = jnp.where(valid, pos, 0)
                sidx_ref[pl.ds(k * L, L)] = plsc.load_gather(src_v, [pos0])
                dvv = plsc.load_gather(dst_v, [pos0])
                didx_ref[pl.ds(k * L, L)] = jnp.where(
                    valid, dvv - tb, ts + (lane & 7))
                return c2

            lax.fori_loop(0, CHUNK // L, ib, 0)

        def scale(off, rows_ref):
            def sb(j, c2):
                jj = jnp.minimum(jnp.zeros((L,), jnp.int32) + (off + j),
                                 cnt - 1)
                pj = plsc.bitcast(plsc.load_gather(lst, [jj]), jnp.int32)
                a_b = plsc.load_gather(ex_v, [pj])
                for r in range(D // L):
                    sl = pl.ds(r * L, L)
                    rows_ref[j, sl] = rows_ref[j, sl] * a_b
                return c2

            lax.fori_loop(0, CHUNK, sb, 0)

        nb = lax.shift_right_logical(cnt + (CHUNK - 1), 6)
        nbe = nb + (nb & 1)

        def pair_body(p, carry):
            @pl.when(p > 0)
            def _drain_prev():
                pltpu.make_async_copy(rows_a, acc.at[didx_a], ssa).wait()
                pltpu.make_async_copy(rows_b, acc.at[didx_b], ssb).wait()

            off_a = (2 * p) * CHUNK
            off_b = off_a + CHUNK
            build_idx(off_a, sidx_a, didx_a)
            ga = pltpu.async_copy(h_hbm.at[sidx_a], rows_a, sga)
            build_idx(off_b, sidx_b, didx_b)
            gb = pltpu.async_copy(h_hbm.at[sidx_b], rows_b, sgb)
            ga.wait()
            scale(off_a, rows_a)
            pltpu.async_copy(rows_a, acc.at[didx_a], ssa, add=True)
            gb.wait()
            scale(off_b, rows_b)
            pltpu.async_copy(rows_b, acc.at[didx_b], ssb, add=True)
            return carry

        lax.fori_loop(0, lax.shift_right_logical(nbe, 1), pair_body, 0)

        @pl.when(nbe > 0)
        def _drain_last():
            pltpu.make_async_copy(rows_a, acc.at[didx_a], ssa).wait()
            pltpu.make_async_copy(rows_b, acc.at[didx_b], ssb).wait()

        plsc.subcore_barrier()

        pltpu.sync_copy(
            acc.at[pl.ds(sid * ROWS_PER_TILE, ROWS_PER_TILE)],
            outp_hbm.at[cid].at[pl.ds(tb + sid * ROWS_PER_TILE,
                                      ROWS_PER_TILE)])

        if ts > NS * ROWS_PER_TILE:
            @pl.when(sid == 0)
            def _dump_tail():
                extra = ts - NS * ROWS_PER_TILE
                pltpu.sync_copy(
                    acc.at[pl.ds(NS * ROWS_PER_TILE, extra)],
                    outp_hbm.at[cid].at[pl.ds(tb + NS * ROWS_PER_TILE,
                                              extra)])

        plsc.subcore_barrier()

        if t < 2:
            lax.fori_loop(0, CHUNK, zrow_body, 0)


def _k4(denc, dst, src, ex, h):
    f = pl.kernel(
        _k4_body,
        out_type=[
            jax.ShapeDtypeStruct((E,), jnp.float32),
            jax.ShapeDtypeStruct((NC, N, D), jnp.float32),
        ],
        mesh=_MESH,
        compiler_params=pltpu.CompilerParams(needs_layout_passes=False),
        scratch_types=[
            pltpu.VMEM((NP,), jnp.float32),
            pltpu.VMEM((LCAP,), jnp.float32),
            pltpu.VMEM((LCAP,), jnp.float32),
            pltpu.VMEM((LCAP,), jnp.float32),
            pltpu.VMEM((EPW,), jnp.int32),
            pltpu.VMEM((EPW,), jnp.int32),
            pltpu.VMEM((EPW,), jnp.float32),
            pltpu.VMEM((CHUNK, D), jnp.float32),
            pltpu.VMEM((CHUNK, D), jnp.float32),
            pltpu.VMEM((CHUNK,), jnp.int32),
            pltpu.VMEM((CHUNK,), jnp.int32),
            pltpu.VMEM((CHUNK,), jnp.int32),
            pltpu.VMEM((CHUNK,), jnp.int32),
            pltpu.SemaphoreType.DMA,
            pltpu.SemaphoreType.DMA,
            pltpu.SemaphoreType.DMA,
            pltpu.SemaphoreType.DMA,
            pltpu.VMEM_SHARED((ACC_ROWS, D), jnp.float32),
        ],
    )
    return f(denc, dst, src, ex, h)


def _k5_body(a_ref, b_ref, o_ref):
    o_ref[...] = a_ref[...] + b_ref[...]


def _k5(p0, p1):
    grid = 10
    bn = N // grid
    return pl.pallas_call(
        _k5_body,
        grid=(grid,),
        in_specs=[
            pl.BlockSpec((bn, D), lambda i: (i, 0)),
            pl.BlockSpec((bn, D), lambda i: (i, 0)),
        ],
        out_specs=pl.BlockSpec((bn, D), lambda i: (i, 0)),
        out_shape=jax.ShapeDtypeStruct((N, D), jnp.float32),
    )(p0, p1)


def kernel(x, edge_index, rnbrw_weights, W, a_src, a_dst):
    src = edge_index[0]
    dst = edge_index[1]
    rn2d = rnbrw_weights.reshape(E // RN_COLS, RN_COLS)
    h, s, b2d = _k1(x, W, a_src, a_dst, rn2d)
    s_flat = s.reshape(2 * N)
    bias = b2d.reshape(E)
    score, mxp = _k2(s_flat, src, dst, bias)
    ex, denp = _k3(mxp, dst, score)
    denc = _k3b(denp)
    alpha, outp = _k4(denc, dst, src, ex, h)
    out = _k5(outp[0], outp[1])
    return out, alpha.reshape(E, 1)


# speedup vs baseline: 14.8401x; 1.0010x over previous
import functools

import jax
import jax.numpy as jnp
from jax import lax
from jax.experimental import pallas as pl
from jax.experimental.pallas import tpu as pltpu
from jax.experimental.pallas import tpu_sc as plsc

N = 10000
E = 320000
D = 128
EPS = 1e-08
NEG_INIT = -3.0e38

NC = 2
NS = 16
NW = NC * NS
EPW = E // NW
L = 16
NP = 10240
SLICE = NP // NS

_MESH = plsc.VectorSubcoreMesh(core_axis_name="c", subcore_axis_name="s")


def _wid():
    return lax.axis_index("s") * NC + lax.axis_index("c")


def _k1_body(x_ref, w_ref, asrc_ref, adst_ref, rn_ref, h_ref, s_ref, b_ref):
    h = lax.dot_general(x_ref[...], w_ref[...],
                        dimension_numbers=(((1,), (1,)), ((), ())),
                        preferred_element_type=jnp.float32)
    h_ref[...] = h
    s_ref[:, 0:1] = jnp.sum(h * asrc_ref[...], axis=1, keepdims=True)
    s_ref[:, 1:2] = jnp.sum(h * adst_ref[...], axis=1, keepdims=True)
    b_ref[...] = jnp.log(rn_ref[...] + EPS)


RN_COLS = 2000


def _k1(x, w, a_src, a_dst, rn2d):
    grid = 10
    bn = N // grid
    br = (E // RN_COLS) // grid
    return pl.pallas_call(
        _k1_body,
        grid=(grid,),
        in_specs=[
            pl.BlockSpec((bn, D), lambda i: (i, 0)),
            pl.BlockSpec((D, D), lambda i: (0, 0)),
            pl.BlockSpec((1, D), lambda i: (0, 0)),
            pl.BlockSpec((1, D), lambda i: (0, 0)),
            pl.BlockSpec((br, RN_COLS), lambda i: (i, 0)),
        ],
        out_specs=[
            pl.BlockSpec((bn, D), lambda i: (i, 0)),
            pl.BlockSpec((bn, 2), lambda i: (i, 0)),
            pl.BlockSpec((br, RN_COLS), lambda i: (i, 0)),
        ],
        out_shape=[
            jax.ShapeDtypeStruct((N, D), jnp.float32),
            jax.ShapeDtypeStruct((N, 2), jnp.float32),
            jax.ShapeDtypeStruct((E // RN_COLS, RN_COLS), jnp.float32),
        ],
    )(x, w, a_src, a_dst, rn2d)


def _k2_body(s_hbm, src_hbm, dst_hbm, bias_hbm, score_hbm, mxp_hbm,
             s_v, src_v, dst_v, bias_v, score_v, mx_v):
    wid = _wid()
    base = wid * EPW
    pltpu.sync_copy(s_hbm, s_v)
    pltpu.sync_copy(src_hbm.at[pl.ds(base, EPW)], src_v)
    pltpu.sync_copy(dst_hbm.at[pl.ds(base, EPW)], dst_v)
    pltpu.sync_copy(bias_hbm.at[pl.ds(base, EPW)], bias_v)

    neg = jnp.full((L,), NEG_INIT, jnp.float32)

    def init_body(i, c):
        mx_v[pl.ds(i * L, L)] = neg
        return c

    lax.fori_loop(0, NP // L, init_body, 0)

    def edge_body(i, c):
        sl = pl.ds(i * L, L)
        sv = src_v[sl]
        dv = dst_v[sl]
        se = plsc.load_gather(s_v, [sv * 2])
        de = plsc.load_gather(s_v, [dv * 2 + 1])
        sc = se + de + bias_v[sl]
        sc = jnp.where(sc >= 0.0, sc, 0.2 * sc)
        score_v[sl] = sc

        def cond(p):
            return jnp.any(p)

        def body(p):
            cur = plsc.load_gather(mx_v, [dv])
            plsc.store_scatter(mx_v, [dv], jnp.maximum(cur, sc), mask=p)
            chk = plsc.load_gather(mx_v, [dv])
            return jnp.logical_and(p, chk < sc)

        lax.while_loop(cond, body, jnp.ones((L,), jnp.bool_))
        return c

    lax.fori_loop(0, EPW // L, edge_body, 0)
    pltpu.sync_copy(score_v, score_hbm.at[pl.ds(base, EPW)])
    pltpu.sync_copy(mx_v, mxp_hbm.at[wid])


def _k2(s_flat, src, dst, bias):
    f = pl.kernel(
        _k2_body,
        out_type=[
            jax.ShapeDtypeStruct((E,), jnp.float32),
            jax.ShapeDtypeStruct((NW, NP), jnp.float32),
        ],
        mesh=_MESH,
        compiler_params=pltpu.CompilerParams(needs_layout_passes=False),
        scratch_types=[
            pltpu.VMEM((2 * N,), jnp.float32),
            pltpu.VMEM((EPW,), jnp.int32),
            pltpu.VMEM((EPW,), jnp.int32),
            pltpu.VMEM((EPW,), jnp.float32),
            pltpu.VMEM((EPW,), jnp.float32),
            pltpu.VMEM((NP,), jnp.float32),
        ],
    )
    return f(s_flat, src, dst, bias)


def _combine(part_hbm, full_v, buf_v, red_v, sp, sem, op):
    sid = lax.axis_index("s")
    off = sid * SLICE
    descs = [pltpu.async_copy(part_hbm.at[r].at[pl.ds(off, SLICE)],
                              buf_v.at[r], sem)
             for r in range(NW)]
    for d in descs:
        d.wait()

    def vec_body(i, c):
        sl = pl.ds(i * L, L)

        def row_body(r, a):
            return op(a, buf_v[r, sl])

        red_v[sl] = lax.fori_loop(1, NW, row_body, buf_v[0, sl])
        return c

    lax.fori_loop(0, SLICE // L, vec_body, 0)
    pltpu.sync_copy(red_v, sp.at[pl.ds(off, SLICE)])
    plsc.subcore_barrier()
    pltpu.sync_copy(sp, full_v)


NR = NP // L


def _k3_body(mxp_hbm, dst_hbm, score_hbm, ex_hbm, denp_hbm,
             mx_v, buf_v, red_v, dst_v, score_v, ex_v, den_v, sem, sp):
    wid = _wid()
    base = wid * EPW
    _combine(mxp_hbm, mx_v, buf_v, red_v, sp, sem, jnp.maximum)
    pltpu.sync_copy(dst_hbm.at[pl.ds(base, EPW)], dst_v)
    pltpu.sync_copy(score_hbm.at[pl.ds(base, EPW)], score_v)

    zero = jnp.zeros((L,), jnp.float32)

    def init_body(i, c):
        den_v[pl.ds(i * L, L)] = zero
        return c

    lax.fori_loop(0, NP // L, init_body, 0)

    def edge_body(i, c):
        sl = pl.ds(i * L, L)
        dv = dst_v[sl]
        ex = jnp.exp(score_v[sl] - plsc.load_gather(mx_v, [dv]))
        ex_v[sl] = ex
        plsc.addupdate_scatter(den_v, [dv], ex)
        return c

    lax.fori_loop(0, EPW // L, edge_body, 0)
    pltpu.sync_copy(ex_v, ex_hbm.at[pl.ds(base, EPW)])
    pltpu.sync_copy(den_v, denp_hbm.at[wid])


def _k3(mxp, dst, score):
    f = pl.kernel(
        _k3_body,
        out_type=[
            jax.ShapeDtypeStruct((E,), jnp.float32),
            jax.ShapeDtypeStruct((NW, NP), jnp.float32),
        ],
        mesh=_MESH,
        compiler_params=pltpu.CompilerParams(needs_layout_passes=False),
        scratch_types=[
            pltpu.VMEM((NP,), jnp.float32),
            pltpu.VMEM((NW, SLICE), jnp.float32),
            pltpu.VMEM((SLICE,), jnp.float32),
            pltpu.VMEM((EPW,), jnp.int32),
            pltpu.VMEM((EPW,), jnp.float32),
            pltpu.VMEM((EPW,), jnp.float32),
            pltpu.VMEM((NP,), jnp.float32),
            pltpu.SemaphoreType.DMA,
            pltpu.VMEM_SHARED((NP,), jnp.float32),
        ],
    )
    return f(mxp, dst, score)


def _k3b_body(denp_hbm, denc_hbm, buf_v, red_v, sem):
    cid = lax.axis_index("c")
    sid = lax.axis_index("s")

    @pl.when(cid == 0)
    def _do():
        off = sid * SLICE
        descs = [pltpu.async_copy(denp_hbm.at[r].at[pl.ds(off, SLICE)],
                                  buf_v.at[r], sem)
                 for r in range(NW)]
        for d in descs:
            d.wait()

        def vec_body(i, c):
            sl = pl.ds(i * L, L)

            def row_body(r, a):
                return a + buf_v[r, sl]

            red_v[sl] = lax.fori_loop(1, NW, row_body, buf_v[0, sl])
            return c

        lax.fori_loop(0, SLICE // L, vec_body, 0)
        pltpu.sync_copy(red_v, denc_hbm.at[pl.ds(off, SLICE)])


def _k3b(denp):
    f = pl.kernel(
        _k3b_body,
        out_type=jax.ShapeDtypeStruct((NP,), jnp.float32),
        mesh=_MESH,
        compiler_params=pltpu.CompilerParams(needs_layout_passes=False),
        scratch_types=[
            pltpu.VMEM((NW, SLICE), jnp.float32),
            pltpu.VMEM((SLICE,), jnp.float32),
            pltpu.SemaphoreType.DMA,
        ],
    )
    return f(denp)


T_BASES = (0, 3328, 6656)
T_SIZES = (3328, 3328, 3344)
ACC_ROWS = max(T_SIZES) + 8
ROWS_PER_TILE = 208
CHUNK = 64
LCAP = EPW + 4 * CHUNK


def _k4_body(denc_hbm, dst_hbm, src_hbm, ex_hbm, h_hbm,
             alpha_hbm, outp_hbm,
             den_v, lst0_v, lst1_v, lst2_v, src_v, dst_v, ex_v,
             rows_a, rows_b, sidx_a, sidx_b, didx_a, didx_b,
             sga, sgb, ssa, ssb, acc):
    cid = lax.axis_index("c")
    sid = lax.axis_index("s")
    wid = sid * NC + cid
    base = wid * EPW
    pltpu.sync_copy(src_hbm.at[pl.ds(base, EPW)], src_v)
    pltpu.sync_copy(dst_hbm.at[pl.ds(base, EPW)], dst_v)
    pltpu.sync_copy(ex_hbm.at[pl.ds(base, EPW)], ex_v)
    pltpu.sync_copy(denc_hbm, den_v)

    def alpha_body(i, c):
        sl = pl.ds(i * L, L)
        dv = dst_v[sl]
        ex_v[sl] = ex_v[sl] / (plsc.load_gather(den_v, [dv]) + EPS)
        return c

    lax.fori_loop(0, EPW // L, alpha_body, 0)
    pltpu.sync_copy(ex_v, alpha_hbm.at[pl.ds(base, EPW)])

    lane = lax.iota(jnp.int32, L)
    lists = (lst0_v, lst1_v, lst2_v)

    def build_body(i, cnts):
        c0, c1, c2 = cnts
        sl = pl.ds(i * L, L)
        dv = dst_v[sl]
        posf = plsc.bitcast(lane + i * L, jnp.float32)
        m0 = dv < T_BASES[1]
        m2 = dv >= T_BASES[2]
        m1 = jnp.logical_and(jnp.logical_not(m0), jnp.logical_not(m2))
        plsc.store_compressed(lst0_v.at[pl.ds(c0, L)], posf, mask=m0)
        plsc.store_compressed(lst1_v.at[pl.ds(c1, L)], posf, mask=m1)
        plsc.store_compressed(lst2_v.at[pl.ds(c2, L)], posf, mask=m2)
        one = jnp.int32(1)
        return (c0 + jnp.sum(jnp.where(m0, one, 0)),
                c1 + jnp.sum(jnp.where(m1, one, 0)),
                c2 + jnp.sum(jnp.where(m2, one, 0)))

    z = jnp.int32(0)
    cnts = lax.fori_loop(0, EPW // L, build_body, (z, z, z))

    zero = jnp.zeros((L,), jnp.float32)

    def zrow_body(j, c):
        for r in range(D // L):
            rows_a[j, pl.ds(r * L, L)] = zero
        return c

    lax.fori_loop(0, CHUNK, zrow_body, 0)

    for t, (tb, ts) in enumerate(zip(T_BASES, T_SIZES)):
        lst = lists[t]
        cnt = cnts[t]
        for zoff in range(0, ROWS_PER_TILE, CHUNK):
            zsz = min(CHUNK, ROWS_PER_TILE - zoff)
            pltpu.sync_copy(
                rows_a.at[pl.ds(0, zsz)],
                acc.at[pl.ds(sid * ROWS_PER_TILE + zoff, zsz)])

        @pl.when(sid == 0)
        def _zero_tail():
            extra = ts + 8 - NS * ROWS_PER_TILE
            pltpu.sync_copy(rows_a.at[pl.ds(0, extra)],
                            acc.at[pl.ds(NS * ROWS_PER_TILE, extra)])

        plsc.subcore_barrier()

        def build_idx(off, sidx_ref, didx_ref):
            def ib(k, c2):
                koff = off + k * L
                pos = plsc.bitcast(lst[pl.ds(koff, L)], jnp.int32)
                valid = (koff + lane) < cnt
                pos0 = jnp.where(valid, pos, 0)
                sidx_ref[pl.ds(k * L, L)] = plsc.load_gather(src_v, [pos0])
                dvv = plsc.load_gather(dst_v, [pos0])
                didx_ref[pl.ds(k * L, L)] = jnp.where(
                    valid, dvv - tb, ts + (lane & 7))
                return c2

            lax.fori_loop(0, CHUNK // L, ib, 0)

        def scale(off, rows_ref):
            def sb(j, c2):
                jj = jnp.minimum(jnp.zeros((L,), jnp.int32) + (off + j),
                                 cnt - 1)
                pj = plsc.bitcast(plsc.load_gather(lst, [jj]), jnp.int32)
                a_b = plsc.load_gather(ex_v, [pj])
                for r in range(D // L):
                    sl = pl.ds(r * L, L)
                    rows_ref[j, sl] = rows_ref[j, sl] * a_b
                return c2

            lax.fori_loop(0, CHUNK, sb, 0)

        nb = lax.shift_right_logical(cnt + (CHUNK - 1), 6)
        nbe = nb + (nb & 1)

        def pair_body(p, carry):
            @pl.when(p > 0)
            def _drain_prev():
                pltpu.make_async_copy(rows_a, acc.at[didx_a], ssa).wait()
                pltpu.make_async_copy(rows_b, acc.at[didx_b], ssb).wait()

            off_a = (2 * p) * CHUNK
            off_b = off_a + CHUNK
            build_idx(off_a, sidx_a, didx_a)
            ga = pltpu.async_copy(h_hbm.at[sidx_a], rows_a, sga)
            build_idx(off_b, sidx_b, didx_b)
            gb = pltpu.async_copy(h_hbm.at[sidx_b], rows_b, sgb)
            ga.wait()
            scale(off_a, rows_a)
            pltpu.async_copy(rows_a, acc.at[didx_a], ssa, add=True)
            gb.wait()
            scale(off_b, rows_b)
            pltpu.async_copy(rows_b, acc.at[didx_b], ssb, add=True)
            return carry

        lax.fori_loop(0, lax.shift_right_logical(nbe, 1), pair_body, 0)

        @pl.when(nbe > 0)
        def _drain_last():
            pltpu.make_async_copy(rows_a, acc.at[didx_a], ssa).wait()
            pltpu.make_async_copy(rows_b, acc.at[didx_b], ssb).wait()

        plsc.subcore_barrier()

        pltpu.sync_copy(
            acc.at[pl.ds(sid * ROWS_PER_TILE, ROWS_PER_TILE)],
            outp_hbm.at[cid].at[pl.ds(tb + sid * ROWS_PER_TILE,
                                      ROWS_PER_TILE)])

        if ts > NS * ROWS_PER_TILE:
            @pl.when(sid == 0)
            def _dump_tail():
                extra = ts - NS * ROWS_PER_TILE
                pltpu.sync_copy(
                    acc.at[pl.ds(NS * ROWS_PER_TILE, extra)],
                    outp_hbm.at[cid].at[pl.ds(tb + NS * ROWS_PER_TILE,
                                              extra)])

        plsc.subcore_barrier()

        if t < 2:
            lax.fori_loop(0, CHUNK, zrow_body, 0)


def _k4(denc, dst, src, ex, h):
    f = pl.kernel(
        _k4_body,
        out_type=[
            jax.ShapeDtypeStruct((E,), jnp.float32),
            jax.ShapeDtypeStruct((NC, N, D), jnp.float32),
        ],
        mesh=_MESH,
        compiler_params=pltpu.CompilerParams(needs_layout_passes=False),
        scratch_types=[
            pltpu.VMEM((NP,), jnp.float32),
            pltpu.VMEM((LCAP,), jnp.float32),
            pltpu.VMEM((LCAP,), jnp.float32),
            pltpu.VMEM((LCAP,), jnp.float32),
            pltpu.VMEM((EPW,), jnp.int32),
            pltpu.VMEM((EPW,), jnp.int32),
            pltpu.VMEM((EPW,), jnp.float32),
            pltpu.VMEM((CHUNK, D), jnp.float32),
            pltpu.VMEM((CHUNK, D), jnp.float32),
            pltpu.VMEM((CHUNK,), jnp.int32),
            pltpu.VMEM((CHUNK,), jnp.int32),
            pltpu.VMEM((CHUNK,), jnp.int32),
            pltpu.VMEM((CHUNK,), jnp.int32),
            pltpu.SemaphoreType.DMA,
            pltpu.SemaphoreType.DMA,
            pltpu.SemaphoreType.DMA,
            pltpu.SemaphoreType.DMA,
            pltpu.VMEM_SHARED((ACC_ROWS, D), jnp.float32),
        ],
    )
    return f(denc, dst, src, ex, h)


def _k5_body(a_ref, b_ref, o_ref):
    o_ref[...] = a_ref[...] + b_ref[...]


def _k5(p0, p1):
    grid = 10
    bn = N // grid
    return pl.pallas_call(
        _k5_body,
        grid=(grid,),
        in_specs=[
            pl.BlockSpec((bn, D), lambda i: (i, 0)),
            pl.BlockSpec((bn, D), lambda i: (i, 0)),
        ],
        out_specs=pl.BlockSpec((bn, D), lambda i: (i, 0)),
        out_shape=jax.ShapeDtypeStruct((N, D), jnp.float32),
    )(p0, p1)


def kernel(x, edge_index, rnbrw_weights, W, a_src, a_dst):
    src = edge_index[0]
    dst = edge_index[1]
    rn2d = rnbrw_weights.reshape(E // RN_COLS, RN_COLS)
    h, s, b2d = _k1(x, W, a_src, a_dst, rn2d)
    s_flat = s.reshape(2 * N)
    bias = b2d.reshape(E)
    score, mxp = _k2(s_flat, src, dst, bias)
    ex, denp = _k3(mxp, dst, score)
    denc = _k3b(denp)
    alpha, outp = _k4(denc, dst, src, ex, h)
    out = _k5(outp[0], outp[1])
    return out, alpha.reshape(E, 1)
